# Initial kernel scaffold; baseline (speedup 1.0000x reference)
import jax
import jax.numpy as jnp
from jax.experimental import pallas as pl


def kernel(x, edge_index, W1, b1, W2, b2):
    raise NotImplementedError("write your pallas kernel here")


# speedup vs baseline: 3.2496x; 3.2496x over previous
import functools

import jax
import jax.numpy as jnp
from jax import lax
from jax.experimental import pallas as pl
from jax.experimental.pallas import tpu as pltpu
from jax.experimental.pallas import tpu_sc as plsc

_NC = 2
_NS = 16
_NW = _NC * _NS
_CHUNK = 128
_NHALF = 2


def _make_agg(n, d, ch):
    assert d % 16 == 0 and ch % (2 * _NHALF) == 0
    n_acc = ((n // (_NS * _CHUNK)) + 1) * (_NS * _CHUNK)
    zc = n_acc // (_NS * _CHUNK)
    n_out = n_acc // _NS
    hc = ch // _NHALF

    mesh = plsc.VectorSubcoreMesh(core_axis_name="c", subcore_axis_name="s")

    @functools.partial(
        pl.kernel,
        out_type=jax.ShapeDtypeStruct((_NC, n_acc, d), jnp.float32),
        mesh=mesh,
        scratch_types=[
            pltpu.VMEM((hc, _CHUNK), jnp.int32),
            pltpu.VMEM((hc, _CHUNK), jnp.int32),
            pltpu.VMEM((2, _CHUNK, d), jnp.float32),
            pltpu.VMEM_SHARED((n_acc, d), jnp.float32),
            pltpu.SemaphoreType.DMA,
            pltpu.SemaphoreType.DMA,
        ],
    )
    def agg(x_hbm, src_hbm, dst_hbm, out_hbm, src_v, dst_v, rows_v,
            acc_sh, sem0, sem1):
        c = lax.axis_index("c")
        s = lax.axis_index("s")
        wid = c * _NS + s

        dlanes = d // 16

        def zbody(i, carry):
            r = i // dlanes
            col = (i % dlanes) * 16
            rows_v[0, r, pl.ds(col, 16)] = jnp.zeros((16,), jnp.float32)
            return carry

        lax.fori_loop(0, _CHUNK * dlanes, zbody, 0)

        zbase = s * (zc * _CHUNK)
        for k in range(zc):
            pltpu.sync_copy(rows_v.at[0],
                            acc_sh.at[pl.ds(zbase + k * _CHUNK, _CHUNK)])
        plsc.subcore_barrier()

        for h in range(_NHALF):
            pltpu.sync_copy(src_hbm.at[wid, h], src_v)
            pltpu.sync_copy(dst_hbm.at[wid, h], dst_v)

            pltpu.async_copy(x_hbm.at[src_v.at[0]], rows_v.at[0], sem0)
            pltpu.async_copy(x_hbm.at[src_v.at[1]], rows_v.at[1], sem1)

            def pair(jj, carry):
                j0 = 2 * jj

                pltpu.make_async_copy(x_hbm.at[src_v.at[j0]], rows_v.at[0],
                                      sem0).wait()
                pltpu.sync_copy(rows_v.at[0], acc_sh.at[dst_v.at[j0]],
                                add=True)

                @pl.when(jj < hc // 2 - 1)
                def _():
                    pltpu.async_copy(x_hbm.at[src_v.at[j0 + 2]],
                                     rows_v.at[0], sem0)

                pltpu.make_async_copy(x_hbm.at[src_v.at[j0 + 1]],
                                      rows_v.at[1], sem1).wait()
                pltpu.sync_copy(rows_v.at[1], acc_sh.at[dst_v.at[j0 + 1]],
                                add=True)

                @pl.when(jj < hc // 2 - 1)
                def _():
                    pltpu.async_copy(x_hbm.at[src_v.at[j0 + 3]],
                                     rows_v.at[1], sem1)

                return carry

            lax.fori_loop(0, hc // 2, pair, 0)
        plsc.subcore_barrier()

        obase = s * n_out
        pltpu.sync_copy(acc_sh.at[pl.ds(obase, n_out)],
                        out_hbm.at[c, pl.ds(obase, n_out)])

    return agg


def _mm_body(p_ref, w_ref, b_ref, o_ref, *, act):
    y = jnp.dot(p_ref[0] + p_ref[1], w_ref[...],
                preferred_element_type=jnp.float32)
    o_ref[...] = act(y + b_ref[...])


def _tc_mm(p, w, b, act, bn=1024):
    _, n, d = p.shape
    co = w.shape[1]
    return pl.pallas_call(
        functools.partial(_mm_body, act=act),
        grid=(n // bn,),
        in_specs=[
            pl.BlockSpec((2, bn, d), lambda i: (0, i, 0)),
            pl.BlockSpec((d, co), lambda i: (0, 0)),
            pl.BlockSpec((1, co), lambda i: (0, 0)),
        ],
        out_specs=pl.BlockSpec((bn, co), lambda i: (i, 0)),
        out_shape=jax.ShapeDtypeStruct((n, co), jnp.float32),
    )(p, w, b)


def kernel(x, edge_index, W1, b1, W2, b2):
    n, d = x.shape
    e = edge_index.shape[1]

    ch = -(-e // (_NW * _CHUNK))
    ch = -(-ch // (2 * _NHALF)) * (2 * _NHALF)
    pad = _NW * ch * _CHUNK - e
    src = jnp.concatenate([edge_index[0], jnp.zeros((pad,), jnp.int32)])
    dst = jnp.concatenate([edge_index[1], jnp.full((pad,), n, jnp.int32)])
    src = src.reshape(_NW, _NHALF, ch // _NHALF, _CHUNK)
    dst = dst.reshape(_NW, _NHALF, ch // _NHALF, _CHUNK)

    agg = _make_agg(n, d, ch)
    a1 = agg(x, src, dst)
    h = _tc_mm(a1, W1, b1.reshape(1, -1), lambda y: jnp.maximum(y, 0.0))
    a2 = agg(h, src, dst)
    out = _tc_mm(a2, W2, b2.reshape(1, -1), jax.nn.sigmoid)
    return out[:n]
